# baseline (speedup 1.0000x reference)
import functools

import jax
import jax.numpy as jnp
from jax import lax
from jax.experimental import pallas as pl
from jax.experimental.pallas import tpu as pltpu
from jax.experimental.pallas import tpu_sc as plsc

BSIZE = 2
NUM_PTS = 10000
IN_C = 128
SPIRAL = 32
OUT_C = 128
PK = OUT_C // 2

NPTS = BSIZE * NUM_PTS
NW = 32
PPW = NPTS // NW
CP = 8
PW = 640
NCHUNK = PW // CP
HALF = NCHUNK // 2
HROWS = HALF * CP
SPH = SPIRAL // 2
ZROWS_H = SPH * NPTS


def _mm_body(x_ref, we_ref, wo_ref, z_ref):
    def half(w_ref):
        acc = lax.dot_general(
            x_ref[...], w_ref[0],
            (((1,), (1,)), ((), ())),
            preferred_element_type=jnp.float32,
        )
        return lax.bitcast_convert_type(
            acc.astype(jnp.bfloat16).astype(jnp.float32), jnp.int32)

    ei = half(we_ref)
    oi = half(wo_ref)
    z_ref[0] = (lax.shift_right_logical(ei, 16) |
                (oi & jnp.int32(-65536)))


def _tc_slot_matmul(xf, we, wo):
    tn = 10000
    ns = we.shape[0]
    return pl.pallas_call(
        _mm_body,
        grid=(NPTS // tn, ns),
        in_specs=[
            pl.BlockSpec((tn, IN_C), lambda t, s: (t, 0)),
            pl.BlockSpec((1, PK, IN_C), lambda t, s: (s, 0, 0)),
            pl.BlockSpec((1, PK, IN_C), lambda t, s: (s, 0, 0)),
        ],
        out_specs=pl.BlockSpec((1, tn, PK), lambda t, s: (s, t, 0)),
        out_shape=jax.ShapeDtypeStruct((ns, NPTS, PK), jnp.int32),
    )(xf, we, wo)


_SC_PARAMS = pltpu.CompilerParams(needs_layout_passes=False,
                                  use_tc_tiling_on_sc=False)


def _sc_phase_body(z_hbm, idx_hbm, part_hbm, bias_hbm, out_hbm,
                   idx_v, rows_a, rows_b, part_a, part_b, out_v, bias_v,
                   sem_a, sem_b, psem_a, psem_b, *, final):
    cid = lax.axis_index("c")
    sid = lax.axis_index("s")
    wid = sid * 2 + cid
    if final:
        pltpu.sync_copy(bias_hbm, bias_v)
    pltpu.sync_copy(idx_hbm.at[wid], idx_v)

    def issue(c, rows_ref, sem, part_ref, psem):
        pltpu.async_copy(z_hbm.at[idx_v.at[c]], rows_ref, sem)
        if final:
            pltpu.async_copy(
                part_hbm.at[pl.ds(wid * (PW * OUT_C) + c * (CP * OUT_C),
                                  CP * OUT_C)],
                part_ref, psem)

    def wait(rows_ref, sem, part_ref, psem):
        pltpu.make_async_copy(z_hbm.at[idx_v.at[0]], rows_ref, sem).wait()
        if final:
            pltpu.make_async_copy(
                part_hbm.at[pl.ds(0, CP * OUT_C)], part_ref, psem).wait()

    two_iota = lax.iota(jnp.int32, 16) * 2
    himask = jnp.full((16,), -65536, dtype=jnp.int32)

    def compute_chunk(c, rows_ref, part_ref):
        obase = (c % HALF) * (CP * OUT_C)
        for jj in range(CP):
            jl = c * CP + jj
            is_edge = (jl == PPW - 1) & ((wid == 15) | (wid == 31))
            scale = jnp.where(is_edge, 0.0, 1.0)
            for g in range(OUT_C // 32):
                sl = pl.ds(g * 16, 16)
                acc_e = jnp.zeros((16,), jnp.float32)
                acc_o = jnp.zeros((16,), jnp.float32)
                for s in range(SPH):
                    w = rows_ref[jj * SPH + s, sl]
                    acc_e = acc_e + plsc.bitcast(w << 16, jnp.float32)
                    acc_o = acc_o + plsc.bitcast(w & himask, jnp.float32)
                col_e = g * 32 + two_iota
                for acc, col in ((acc_e, col_e), (acc_o, col_e + 1)):
                    if final:
                        v = (acc + plsc.load_gather(part_ref,
                                                    [jj * OUT_C + col])
                             + plsc.load_gather(bias_v, [col]))
                        v = jnp.where(v > 0.0, v, jnp.exp(v) - 1.0) * scale
                    else:
                        v = acc
                    plsc.store_scatter(
                        out_v, [obase + jj * OUT_C + col], v)

    issue(0, rows_a, sem_a, part_a, psem_a)

    def body(i, carry):
        c0 = 2 * i
        issue(c0 + 1, rows_b, sem_b, part_b, psem_b)
        wait(rows_a, sem_a, part_a, psem_a)
        compute_chunk(c0, rows_a, part_a)

        @pl.when(i < NCHUNK // 2 - 1)
        def _prefetch():
            issue(c0 + 2, rows_a, sem_a, part_a, psem_a)

        wait(rows_b, sem_b, part_b, psem_b)
        compute_chunk(c0 + 1, rows_b, part_b)

        @pl.when(i == HALF // 2 - 1)
        def _flush0():
            pltpu.sync_copy(
                out_v, out_hbm.at[pl.ds(wid * (PW * OUT_C), HROWS * OUT_C)])

        @pl.when(i == NCHUNK // 2 - 1)
        def _flush1():
            pltpu.sync_copy(
                out_v,
                out_hbm.at[pl.ds(wid * (PW * OUT_C) + HROWS * OUT_C,
                                 HROWS * OUT_C)])

        return carry

    lax.fori_loop(0, NCHUNK // 2, body, 0)


def _sc_scratch():
    return [
        pltpu.VMEM((NCHUNK, CP * SPH), jnp.int32),
        pltpu.VMEM((CP * SPH, PK), jnp.int32),
        pltpu.VMEM((CP * SPH, PK), jnp.int32),
        pltpu.VMEM((CP * OUT_C,), jnp.float32),
        pltpu.VMEM((CP * OUT_C,), jnp.float32),
        pltpu.VMEM((HROWS * OUT_C,), jnp.float32),
        pltpu.VMEM((OUT_C,), jnp.float32),
        pltpu.SemaphoreType.DMA,
        pltpu.SemaphoreType.DMA,
        pltpu.SemaphoreType.DMA,
        pltpu.SemaphoreType.DMA,
    ]


@functools.partial(
    pl.kernel,
    out_type=jax.ShapeDtypeStruct((NW * PW * OUT_C,), jnp.float32),
    mesh=plsc.VectorSubcoreMesh(core_axis_name="c", subcore_axis_name="s"),
    compiler_params=_SC_PARAMS,
    scratch_types=_sc_scratch(),
)
def _sc_partial(z_hbm, idx_hbm, out_hbm,
                idx_v, rows_a, rows_b, part_a, part_b, out_v, bias_v,
                sem_a, sem_b, psem_a, psem_b):
    _sc_phase_body(z_hbm, idx_hbm, None, None, out_hbm,
                   idx_v, rows_a, rows_b, part_a, part_b, out_v, bias_v,
                   sem_a, sem_b, psem_a, psem_b, final=False)


@functools.partial(
    pl.kernel,
    out_type=jax.ShapeDtypeStruct((NW * PW * OUT_C,), jnp.float32),
    mesh=plsc.VectorSubcoreMesh(core_axis_name="c", subcore_axis_name="s"),
    compiler_params=_SC_PARAMS,
    scratch_types=_sc_scratch(),
)
def _sc_final(z_hbm, idx_hbm, part_hbm, bias_hbm, out_hbm,
              idx_v, rows_a, rows_b, part_a, part_b, out_v, bias_v,
              sem_a, sem_b, psem_a, psem_b):
    _sc_phase_body(z_hbm, idx_hbm, part_hbm, bias_hbm, out_hbm,
                   idx_v, rows_a, rows_b, part_a, part_b, out_v, bias_v,
                   sem_a, sem_b, psem_a, psem_b, final=True)


def kernel(x, spiral_adj, W, b):
    xf = x.reshape(NPTS, IN_C).astype(jnp.bfloat16)
    wr = (W.reshape(OUT_C, SPIRAL, IN_C).transpose(1, 0, 2)
          .astype(jnp.bfloat16))
    we = wr[:, 0::2, :]
    wo = wr[:, 1::2, :]

    adj = spiral_adj.astype(jnp.int32)
    idx = (adj
           + (jnp.arange(BSIZE, dtype=jnp.int32) * NUM_PTS)[:, None, None]
           + ((jnp.arange(SPIRAL, dtype=jnp.int32) % SPH) * NPTS)
           [None, None, :])
    idx = idx.reshape(NW, PPW, SPIRAL)
    idx = jnp.pad(idx, ((0, 0), (0, PW - PPW), (0, 0)))
    idx1 = idx[:, :, :SPH].reshape(NW, NCHUNK, CP * SPH)
    idx2 = idx[:, :, SPH:].reshape(NW, NCHUNK, CP * SPH)

    z1 = _tc_slot_matmul(xf, we[:SPH], wo[:SPH]).reshape(ZROWS_H, PK)
    part = _sc_partial(z1, idx1)
    z2 = _tc_slot_matmul(xf, we[SPH:], wo[SPH:]).reshape(ZROWS_H, PK)
    buf = _sc_final(z2, idx2, part, b)
    out = buf.reshape(NW, PW, OUT_C)[:, :PPW, :]
    return out.reshape(BSIZE, NUM_PTS, OUT_C)


# speedup vs baseline: 1.0268x; 1.0268x over previous
import functools

import jax
import jax.numpy as jnp
from jax import lax
from jax.experimental import pallas as pl
from jax.experimental.pallas import tpu as pltpu
from jax.experimental.pallas import tpu_sc as plsc

BSIZE = 2
NUM_PTS = 10000
IN_C = 128
SPIRAL = 32
OUT_C = 128
PK = OUT_C // 2

NPTS = BSIZE * NUM_PTS
NW = 32
PPW = NPTS // NW
CP = 4
PW = 640
NCHUNK = PW // CP
HALF = NCHUNK // 2
HROWS = HALF * CP
SPH = SPIRAL // 2
ZROWS_H = SPH * NPTS


def _mm_body(x_ref, we_ref, wo_ref, z_ref):
    def half(w_ref):
        acc = lax.dot_general(
            x_ref[...], w_ref[0],
            (((1,), (1,)), ((), ())),
            preferred_element_type=jnp.float32,
        )
        return lax.bitcast_convert_type(
            acc.astype(jnp.bfloat16).astype(jnp.float32), jnp.int32)

    ei = half(we_ref)
    oi = half(wo_ref)
    z_ref[0] = (lax.shift_right_logical(ei, 16) |
                (oi & jnp.int32(-65536)))


def _tc_slot_matmul(xf, we, wo):
    tn = 10000
    ns = we.shape[0]
    return pl.pallas_call(
        _mm_body,
        grid=(NPTS // tn, ns),
        in_specs=[
            pl.BlockSpec((tn, IN_C), lambda t, s: (t, 0)),
            pl.BlockSpec((1, PK, IN_C), lambda t, s: (s, 0, 0)),
            pl.BlockSpec((1, PK, IN_C), lambda t, s: (s, 0, 0)),
        ],
        out_specs=pl.BlockSpec((1, tn, PK), lambda t, s: (s, t, 0)),
        out_shape=jax.ShapeDtypeStruct((ns, NPTS, PK), jnp.int32),
    )(xf, we, wo)


_SC_PARAMS = pltpu.CompilerParams(needs_layout_passes=False,
                                  use_tc_tiling_on_sc=False)


def _sc_phase_body(z_hbm, idx_hbm, part_hbm, bias_hbm, out_hbm,
                   idx_v, rows_a, rows_b, part_a, part_b, out_v, bias_v,
                   sem_a, sem_b, psem_a, psem_b, *, final):
    cid = lax.axis_index("c")
    sid = lax.axis_index("s")
    wid = sid * 2 + cid
    if final:
        pltpu.sync_copy(bias_hbm, bias_v)
    pltpu.sync_copy(idx_hbm.at[wid], idx_v)

    def issue(c, rows_ref, sem, part_ref, psem):
        pltpu.async_copy(z_hbm.at[idx_v.at[c]], rows_ref, sem)
        if final:
            pltpu.async_copy(
                part_hbm.at[pl.ds(wid * (PW * OUT_C) + c * (CP * OUT_C),
                                  CP * OUT_C)],
                part_ref, psem)

    def wait(rows_ref, sem, part_ref, psem):
        pltpu.make_async_copy(z_hbm.at[idx_v.at[0]], rows_ref, sem).wait()
        if final:
            pltpu.make_async_copy(
                part_hbm.at[pl.ds(0, CP * OUT_C)], part_ref, psem).wait()

    two_iota = lax.iota(jnp.int32, 16) * 2
    himask = jnp.full((16,), -65536, dtype=jnp.int32)

    def compute_chunk(c, rows_ref, part_ref):
        obase = (c % HALF) * (CP * OUT_C)
        for jj in range(CP):
            jl = c * CP + jj
            is_edge = (jl == PPW - 1) & ((wid == 15) | (wid == 31))
            scale = jnp.where(is_edge, 0.0, 1.0)
            for g in range(OUT_C // 32):
                sl = pl.ds(g * 16, 16)
                acc_e = jnp.zeros((16,), jnp.float32)
                acc_o = jnp.zeros((16,), jnp.float32)
                for s in range(SPH):
                    w = rows_ref[jj * SPH + s, sl]
                    acc_e = acc_e + plsc.bitcast(w << 16, jnp.float32)
                    acc_o = acc_o + plsc.bitcast(w & himask, jnp.float32)
                col_e = g * 32 + two_iota
                for acc, col in ((acc_e, col_e), (acc_o, col_e + 1)):
                    if final:
                        v = (acc + plsc.load_gather(part_ref,
                                                    [jj * OUT_C + col])
                             + plsc.load_gather(bias_v, [col]))
                        v = jnp.where(v > 0.0, v, jnp.exp(v) - 1.0) * scale
                    else:
                        v = acc
                    plsc.store_scatter(
                        out_v, [obase + jj * OUT_C + col], v)

    issue(0, rows_a, sem_a, part_a, psem_a)

    def body(i, carry):
        c0 = 2 * i
        issue(c0 + 1, rows_b, sem_b, part_b, psem_b)
        wait(rows_a, sem_a, part_a, psem_a)
        compute_chunk(c0, rows_a, part_a)

        @pl.when(i < NCHUNK // 2 - 1)
        def _prefetch():
            issue(c0 + 2, rows_a, sem_a, part_a, psem_a)

        wait(rows_b, sem_b, part_b, psem_b)
        compute_chunk(c0 + 1, rows_b, part_b)

        @pl.when(i == HALF // 2 - 1)
        def _flush0():
            pltpu.sync_copy(
                out_v, out_hbm.at[pl.ds(wid * (PW * OUT_C), HROWS * OUT_C)])

        @pl.when(i == NCHUNK // 2 - 1)
        def _flush1():
            pltpu.sync_copy(
                out_v,
                out_hbm.at[pl.ds(wid * (PW * OUT_C) + HROWS * OUT_C,
                                 HROWS * OUT_C)])

        return carry

    lax.fori_loop(0, NCHUNK // 2, body, 0)


def _sc_scratch():
    return [
        pltpu.VMEM((NCHUNK, CP * SPH), jnp.int32),
        pltpu.VMEM((CP * SPH, PK), jnp.int32),
        pltpu.VMEM((CP * SPH, PK), jnp.int32),
        pltpu.VMEM((CP * OUT_C,), jnp.float32),
        pltpu.VMEM((CP * OUT_C,), jnp.float32),
        pltpu.VMEM((HROWS * OUT_C,), jnp.float32),
        pltpu.VMEM((OUT_C,), jnp.float32),
        pltpu.SemaphoreType.DMA,
        pltpu.SemaphoreType.DMA,
        pltpu.SemaphoreType.DMA,
        pltpu.SemaphoreType.DMA,
    ]


@functools.partial(
    pl.kernel,
    out_type=jax.ShapeDtypeStruct((NW * PW * OUT_C,), jnp.float32),
    mesh=plsc.VectorSubcoreMesh(core_axis_name="c", subcore_axis_name="s"),
    compiler_params=_SC_PARAMS,
    scratch_types=_sc_scratch(),
)
def _sc_partial(z_hbm, idx_hbm, out_hbm,
                idx_v, rows_a, rows_b, part_a, part_b, out_v, bias_v,
                sem_a, sem_b, psem_a, psem_b):
    _sc_phase_body(z_hbm, idx_hbm, None, None, out_hbm,
                   idx_v, rows_a, rows_b, part_a, part_b, out_v, bias_v,
                   sem_a, sem_b, psem_a, psem_b, final=False)


@functools.partial(
    pl.kernel,
    out_type=jax.ShapeDtypeStruct((NW * PW * OUT_C,), jnp.float32),
    mesh=plsc.VectorSubcoreMesh(core_axis_name="c", subcore_axis_name="s"),
    compiler_params=_SC_PARAMS,
    scratch_types=_sc_scratch(),
)
def _sc_final(z_hbm, idx_hbm, part_hbm, bias_hbm, out_hbm,
              idx_v, rows_a, rows_b, part_a, part_b, out_v, bias_v,
              sem_a, sem_b, psem_a, psem_b):
    _sc_phase_body(z_hbm, idx_hbm, part_hbm, bias_hbm, out_hbm,
                   idx_v, rows_a, rows_b, part_a, part_b, out_v, bias_v,
                   sem_a, sem_b, psem_a, psem_b, final=True)


def kernel(x, spiral_adj, W, b):
    xf = x.reshape(NPTS, IN_C).astype(jnp.bfloat16)
    wr = (W.reshape(OUT_C, SPIRAL, IN_C).transpose(1, 0, 2)
          .astype(jnp.bfloat16))
    we = wr[:, 0::2, :]
    wo = wr[:, 1::2, :]

    adj = spiral_adj.astype(jnp.int32)
    idx = (adj
           + (jnp.arange(BSIZE, dtype=jnp.int32) * NUM_PTS)[:, None, None]
           + ((jnp.arange(SPIRAL, dtype=jnp.int32) % SPH) * NPTS)
           [None, None, :])
    idx = idx.reshape(NW, PPW, SPIRAL)
    idx = jnp.pad(idx, ((0, 0), (0, PW - PPW), (0, 0)))
    idx1 = idx[:, :, :SPH].reshape(NW, NCHUNK, CP * SPH)
    idx2 = idx[:, :, SPH:].reshape(NW, NCHUNK, CP * SPH)

    z1 = _tc_slot_matmul(xf, we[:SPH], wo[:SPH]).reshape(ZROWS_H, PK)
    part = _sc_partial(z1, idx1)
    z2 = _tc_slot_matmul(xf, we[SPH:], wo[SPH:]).reshape(ZROWS_H, PK)
    buf = _sc_final(z2, idx2, part, b)
    out = buf.reshape(NW, PW, OUT_C)[:, :PPW, :]
    return out.reshape(BSIZE, NUM_PTS, OUT_C)
